# Initial kernel scaffold; baseline (speedup 1.0000x reference)
import functools

import jax
import jax.numpy as jnp
from jax import lax
from jax.experimental import pallas as pl
from jax.experimental.pallas import tpu as pltpu
from jax.experimental.pallas import tpu_sc as plsc

N_NODES = 10000
NROWS = 10240
R = 512
NB = NROWS // R
F = 128
DCOL = 144
NC, NS, L = 2, 16, 16
NW = NC * NS
CH = 128
E_PAD = 331776
EPW = E_PAD // NW
ROWS_PER_SUB = NROWS // NS

_NEG = -1e30


def _a_body(heads, x_ref, w_ref, asrc_ref, adst_ref,
            tab_ref, as_ref, ad_ref, sh_ref):
    i = pl.program_id(0)
    h1 = jnp.dot(x_ref[...], w_ref[...], preferred_element_type=jnp.float32)
    rows = i * R + lax.broadcasted_iota(jnp.int32, (R,), 0)
    valid = rows < N_NODES
    extcol = jnp.where(
        lax.broadcasted_iota(jnp.int32, (R, DCOL - F), 1) == 0, 1.0, 0.0)
    as_list, ad_list = [], []
    for h in range(heads):
        hb = h1[:, h * F:(h + 1) * F]
        tab_ref[h] = jnp.concatenate([hb, extcol], axis=1)
        as_list.append(jnp.sum(hb * asrc_ref[h][None, :], axis=1))
        ad_list.append(jnp.sum(hb * adst_ref[h][None, :], axis=1))
    asb = jnp.stack(as_list, axis=0)
    adb = jnp.stack(ad_list, axis=0)
    validb = valid[None, :]
    asb = jnp.where(validb, asb, _NEG)
    adb = jnp.where(validb, adb, _NEG)
    as_ref[...] = asb
    ad_ref[...] = adb
    blk_as = jnp.max(asb, axis=1)
    blk_ad = jnp.max(adb, axis=1)
    cols = lax.broadcasted_iota(jnp.int32, (heads, 16), 1)
    cur = jnp.where(cols < 8, blk_as[:, None], blk_ad[:, None])
    prev = jnp.where(i == 0, jnp.full((heads, 16), _NEG, jnp.float32),
                     sh_ref[...])
    sh_ref[...] = jnp.maximum(prev, cur)


def _make_a_call(heads, in_ch):
    body = functools.partial(_a_body, heads)
    return pl.pallas_call(
        body,
        grid=(NB,),
        in_specs=[
            pl.BlockSpec((R, in_ch), lambda i: (i, 0)),
            pl.BlockSpec((in_ch, heads * F), lambda i: (0, 0)),
            pl.BlockSpec((heads, F), lambda i: (0, 0)),
            pl.BlockSpec((heads, F), lambda i: (0, 0)),
        ],
        out_specs=[
            pl.BlockSpec((heads, R, DCOL), lambda i: (0, i, 0)),
            pl.BlockSpec((heads, R), lambda i: (0, i)),
            pl.BlockSpec((heads, R), lambda i: (0, i)),
            pl.BlockSpec((heads, 16), lambda i: (0, 0)),
        ],
        out_shape=[
            jax.ShapeDtypeStruct((heads, NROWS, DCOL), jnp.float32),
            jax.ShapeDtypeStruct((heads, NROWS), jnp.float32),
            jax.ShapeDtypeStruct((heads, NROWS), jnp.float32),
            jax.ShapeDtypeStruct((heads, 16), jnp.float32),
        ],
    )


def _make_sc_call(heads):
    mesh = plsc.VectorSubcoreMesh(core_axis_name="c", subcore_axis_name="s")

    @functools.partial(
        pl.kernel,
        out_type=jax.ShapeDtypeStruct((NC, heads, NROWS, DCOL), jnp.float32),
        mesh=mesh,
        scratch_types=[
            pltpu.VMEM((EPW,), jnp.int32),
            pltpu.VMEM((EPW,), jnp.int32),
            pltpu.VMEM((CH,), jnp.int32),
            pltpu.VMEM((CH,), jnp.int32),
            pltpu.VMEM((CH,), jnp.float32),
            pltpu.VMEM((CH, DCOL), jnp.float32),
            pltpu.VMEM((NROWS,), jnp.float32),
            pltpu.VMEM((NROWS,), jnp.float32),
            pltpu.VMEM((16,), jnp.float32),
            pltpu.VMEM_SHARED((NROWS, DCOL), jnp.float32),
            pltpu.SemaphoreType.DMA,
        ],
    )
    def sc_msg(tab, srcp, dstp, as_hbm, ad_hbm, sh_hbm, out,
               src_v, dst_v, gidx_v, didx_v, p_v, rows_v,
               as_v, ad_v, sh_v, acc, sem):
        cid = lax.axis_index("c")
        sid = lax.axis_index("s")
        wid = cid * NS + sid
        ebase = wid * EPW
        rbase = sid * ROWS_PER_SUB
        pltpu.sync_copy(srcp.at[pl.ds(ebase, EPW)], src_v)
        pltpu.sync_copy(dstp.at[pl.ds(ebase, EPW)], dst_v)

        for h in range(heads):
            @pl.loop(0, CH)
            def _(k):
                @pl.loop(0, DCOL, step=L)
                def _(j):
                    rows_v[k, pl.ds(j, L)] = jnp.zeros((L,), jnp.float32)

            @pl.loop(0, ROWS_PER_SUB, step=CH)
            def _(r):
                pltpu.sync_copy(rows_v, acc.at[pl.ds(rbase + r, CH)])

            pltpu.sync_copy(as_hbm.at[h], as_v)
            pltpu.sync_copy(ad_hbm.at[h], ad_v)
            pltpu.sync_copy(sh_hbm.at[h], sh_v)
            plsc.subcore_barrier()

            @pl.loop(0, EPW, step=CH)
            def _(c):
                shv = sh_v[...]

                @pl.loop(0, CH, step=L)
                def _(g):
                    sv = src_v[pl.ds(c + g, L)]
                    dv = dst_v[pl.ds(c + g, L)]
                    e = plsc.load_gather(as_v, [sv]) + plsc.load_gather(ad_v, [dv])
                    e = jnp.maximum(e, e * 0.2)
                    p_v[pl.ds(g, L)] = jnp.exp(e - shv)
                    gidx_v[pl.ds(g, L)] = sv + h * NROWS
                    didx_v[pl.ds(g, L)] = dv

                pltpu.async_copy(tab.at[gidx_v], rows_v, sem).wait()

                @pl.loop(0, CH)
                def _(k):
                    pb = jnp.full((L,), p_v[k], jnp.float32)

                    @pl.loop(0, DCOL, step=L)
                    def _(j):
                        rows_v[k, pl.ds(j, L)] = rows_v[k, pl.ds(j, L)] * pb

                pltpu.sync_copy(rows_v, acc.at[didx_v], add=True)

            plsc.subcore_barrier()
            pltpu.sync_copy(acc.at[pl.ds(rbase, ROWS_PER_SUB)],
                            out.at[cid, h, pl.ds(rbase, ROWS_PER_SUB)])

    return sc_msg


def _e1_body(o_ref, b_ref, out_ref):
    o = o_ref[0, 0] + o_ref[1, 0]
    feats = o[:, :F]
    den = jnp.maximum(o[:, F:F + 1], 1e-30)
    r = feats / den + b_ref[0, 0][None, :]
    out_ref[...] = jnp.maximum(r, 0.0)


_e1_call = pl.pallas_call(
    _e1_body,
    grid=(NB, 4),
    in_specs=[
        pl.BlockSpec((NC, 1, R, DCOL), lambda i, h: (0, h, i, 0)),
        pl.BlockSpec((1, 1, F), lambda i, h: (h, 0, 0)),
    ],
    out_specs=pl.BlockSpec((R, F), lambda i, h: (i, h)),
    out_shape=jax.ShapeDtypeStruct((NROWS, 4 * F), jnp.float32),
)


def _e2_body(o_ref, b_ref, wc_ref, y_ref):
    o = o_ref[0, 0] + o_ref[1, 0]
    den = jnp.maximum(o[:, F:F + 1], 1e-30)
    h2 = o[:, :F] / den + b_ref[0][None, :]
    y = jnp.sum(h2 * wc_ref[0][None, :], axis=1)
    y_ref[...] = jnp.broadcast_to(y[None, :], (8, R))


_e2_call = pl.pallas_call(
    _e2_body,
    grid=(NB,),
    in_specs=[
        pl.BlockSpec((NC, 1, R, DCOL), lambda i: (0, 0, i, 0)),
        pl.BlockSpec((1, F), lambda i: (0, 0)),
        pl.BlockSpec((1, F), lambda i: (0, 0)),
    ],
    out_specs=pl.BlockSpec((8, R), lambda i: (0, i)),
    out_shape=jax.ShapeDtypeStruct((8, NROWS), jnp.float32),
)

_a1_call = _make_a_call(4, 128)
_a2_call = _make_a_call(1, 512)
_sc4_call = _make_sc_call(4)
_sc1_call = _make_sc_call(1)


def _shift_vec(sh_raw):
    return jnp.broadcast_to(
        jnp.maximum(sh_raw[:, 0] + sh_raw[:, 8], 0.0)[:, None],
        (sh_raw.shape[0], 16))


@jax.jit
def kernel(x, edge_index, W1, att_src1, att_dst1, b1,
           W2, att_src2, att_dst2, b2, Wc, bc):
    n = x.shape[0]
    xp = jnp.pad(x, ((0, NROWS - n), (0, 0)))
    loop = jnp.arange(n, dtype=jnp.int32)
    src = jnp.concatenate([edge_index[0].astype(jnp.int32), loop])
    dst = jnp.concatenate([edge_index[1].astype(jnp.int32), loop])
    pad_e = E_PAD - src.shape[0]
    srcp = jnp.concatenate([src, jnp.full((pad_e,), n, jnp.int32)])
    dstp = jnp.concatenate([dst, jnp.zeros((pad_e,), jnp.int32)])

    tab1, as1, ad1, shraw1 = _a1_call(xp, W1, att_src1, att_dst1)
    o1 = _sc4_call(tab1.reshape(4 * NROWS, DCOL), srcp, dstp,
                   as1, ad1, _shift_vec(shraw1))
    h2in = _e1_call(o1, b1.reshape(4, 1, F))

    tab2, as2, ad2, shraw2 = _a2_call(h2in, W2, att_src2, att_dst2)
    o2 = _sc1_call(tab2.reshape(NROWS, DCOL), srcp, dstp,
                   as2, ad2, _shift_vec(shraw2))
    y8 = _e2_call(o2, b2.reshape(1, F), Wc.reshape(1, F))
    return y8[0, :n] + bc[0]


# speedup vs baseline: 14.3156x; 14.3156x over previous
import functools

import jax
import jax.numpy as jnp
from jax import lax
from jax.experimental import pallas as pl
from jax.experimental.pallas import tpu as pltpu
from jax.experimental.pallas import tpu_sc as plsc

N_NODES = 10000
NROWS = 10240
R = 512
NB = NROWS // R
F = 128
FG = 64
DCOL = 80
NC, NS, L = 2, 16, 16
NW = NC * NS
CH = 128
E_PAD = 331776
EPW = E_PAD // NW
ROWS_PER_SUB = NROWS // NS

_NEG = -1e30


def _a_body(heads, x_ref, w_ref, asrc_ref, adst_ref,
            tab_ref, as_ref, ad_ref, sh_ref):
    i = pl.program_id(0)
    h1 = jnp.dot(x_ref[...], w_ref[...], preferred_element_type=jnp.float32)
    rows = i * R + lax.broadcasted_iota(jnp.int32, (R,), 0)
    valid = rows < N_NODES
    onescol = jnp.where(
        lax.broadcasted_iota(jnp.int32, (R, DCOL - FG), 1) == 0, 1.0, 0.0)
    zerocol = jnp.zeros((R, DCOL - FG), jnp.float32)
    as_list, ad_list = [], []
    for h in range(heads):
        hb = h1[:, h * F:(h + 1) * F]
        tab_ref[2 * h] = jnp.concatenate([hb[:, :FG], onescol], axis=1)
        tab_ref[2 * h + 1] = jnp.concatenate([hb[:, FG:], zerocol], axis=1)
        as_list.append(jnp.sum(hb * asrc_ref[h][None, :], axis=1))
        ad_list.append(jnp.sum(hb * adst_ref[h][None, :], axis=1))
    asb = jnp.stack(as_list, axis=0)
    adb = jnp.stack(ad_list, axis=0)
    validb = valid[None, :]
    asb = jnp.where(validb, asb, _NEG)
    adb = jnp.where(validb, adb, _NEG)
    as_ref[...] = asb
    ad_ref[...] = adb
    blk_as = jnp.max(asb, axis=1)
    blk_ad = jnp.max(adb, axis=1)
    cols = lax.broadcasted_iota(jnp.int32, (heads, 16), 1)
    cur = jnp.where(cols < 8, blk_as[:, None], blk_ad[:, None])
    prev = jnp.where(i == 0, jnp.full((heads, 16), _NEG, jnp.float32),
                     sh_ref[...])
    sh_ref[...] = jnp.maximum(prev, cur)


def _make_a_call(heads, in_ch):
    body = functools.partial(_a_body, heads)
    return pl.pallas_call(
        body,
        grid=(NB,),
        in_specs=[
            pl.BlockSpec((R, in_ch), lambda i: (i, 0)),
            pl.BlockSpec((in_ch, heads * F), lambda i: (0, 0)),
            pl.BlockSpec((heads, F), lambda i: (0, 0)),
            pl.BlockSpec((heads, F), lambda i: (0, 0)),
        ],
        out_specs=[
            pl.BlockSpec((2 * heads, R, DCOL), lambda i: (0, i, 0)),
            pl.BlockSpec((heads, R), lambda i: (0, i)),
            pl.BlockSpec((heads, R), lambda i: (0, i)),
            pl.BlockSpec((heads, 16), lambda i: (0, 0)),
        ],
        out_shape=[
            jax.ShapeDtypeStruct((2 * heads, NROWS, DCOL), jnp.float32),
            jax.ShapeDtypeStruct((heads, NROWS), jnp.float32),
            jax.ShapeDtypeStruct((heads, NROWS), jnp.float32),
            jax.ShapeDtypeStruct((heads, 16), jnp.float32),
        ],
    )


def _make_sc_call(heads):
    mesh = plsc.VectorSubcoreMesh(core_axis_name="c", subcore_axis_name="s")
    cp = pltpu.CompilerParams(needs_layout_passes=False,
                              use_tc_tiling_on_sc=False)

    @functools.partial(
        pl.kernel,
        compiler_params=cp,
        out_type=jax.ShapeDtypeStruct((NC, 2 * heads, NROWS, DCOL),
                                      jnp.float32),
        mesh=mesh,
        scratch_types=[
            pltpu.VMEM((EPW,), jnp.int32),
            pltpu.VMEM((EPW,), jnp.int32),
            pltpu.VMEM((CH,), jnp.int32),
            pltpu.VMEM((CH,), jnp.int32),
            pltpu.VMEM((CH,), jnp.float32),
            pltpu.VMEM((CH, DCOL), jnp.float32),
            pltpu.VMEM((NROWS,), jnp.float32),
            pltpu.VMEM((NROWS,), jnp.float32),
            pltpu.VMEM((16,), jnp.float32),
            pltpu.VMEM_SHARED((NROWS, DCOL), jnp.float32),
            pltpu.SemaphoreType.DMA,
        ],
    )
    def sc_msg(tab, srcp, dstp, as_hbm, ad_hbm, sh_hbm, out,
               src_v, dst_v, gidx_v, didx_v, p_v, rows_v,
               as_v, ad_v, sh_v, acc, sem):
        cid = lax.axis_index("c")
        sid = lax.axis_index("s")
        wid = cid * NS + sid
        ebase = wid * EPW
        rbase = sid * ROWS_PER_SUB
        pltpu.sync_copy(srcp.at[pl.ds(ebase, EPW)], src_v)
        pltpu.sync_copy(dstp.at[pl.ds(ebase, EPW)], dst_v)

        for ph in range(2 * heads):
            h = ph // 2
            @pl.loop(0, CH)
            def _(k):
                @pl.loop(0, DCOL, step=L)
                def _(j):
                    rows_v[k, pl.ds(j, L)] = jnp.zeros((L,), jnp.float32)

            @pl.loop(0, ROWS_PER_SUB, step=CH)
            def _(r):
                pltpu.sync_copy(rows_v, acc.at[pl.ds(rbase + r, CH)])

            pltpu.sync_copy(as_hbm.at[h], as_v)
            pltpu.sync_copy(ad_hbm.at[h], ad_v)
            pltpu.sync_copy(sh_hbm.at[h], sh_v)
            plsc.subcore_barrier()

            @pl.loop(0, EPW, step=CH)
            def _(c):
                shv = sh_v[...]

                @pl.loop(0, CH, step=L)
                def _(g):
                    sv = src_v[pl.ds(c + g, L)]
                    dv = dst_v[pl.ds(c + g, L)]
                    e = (plsc.load_gather(as_v, [sv])
                         + plsc.load_gather(ad_v, [dv]))
                    e = jnp.maximum(e, e * 0.2)
                    p_v[pl.ds(g, L)] = jnp.exp(e - shv)
                    gidx_v[pl.ds(g, L)] = sv + ph * NROWS
                    didx_v[pl.ds(g, L)] = dv

                pltpu.async_copy(tab.at[gidx_v], rows_v, sem).wait()

                @pl.loop(0, CH, step=L)
                def _(g):
                    pv = p_v[pl.ds(g, L)]
                    for j in range(L):
                        pb = jnp.full((L,), pv[j], jnp.float32)
                        for jc in range(0, DCOL, L):
                            rows_v[g + j, pl.ds(jc, L)] = (
                                rows_v[g + j, pl.ds(jc, L)] * pb)

                pltpu.sync_copy(rows_v, acc.at[didx_v], add=True)

            plsc.subcore_barrier()
            pltpu.sync_copy(acc.at[pl.ds(rbase, ROWS_PER_SUB)],
                            out.at[cid, ph, pl.ds(rbase, ROWS_PER_SUB)])

    return sc_msg


def _e1_body(o_ref, b_ref, out_ref):
    g0 = o_ref[0, 0] + o_ref[1, 0]
    g1 = o_ref[0, 1] + o_ref[1, 1]
    feats = jnp.concatenate([g0[:, :FG], g1[:, :FG]], axis=1)
    den = jnp.maximum(g0[:, FG:FG + 1], 1e-30)
    r = feats / den + b_ref[0, 0][None, :]
    out_ref[...] = jnp.maximum(r, 0.0)


_e1_call = pl.pallas_call(
    _e1_body,
    grid=(NB, 4),
    in_specs=[
        pl.BlockSpec((NC, 2, R, DCOL), lambda i, h: (0, h, i, 0)),
        pl.BlockSpec((1, 1, F), lambda i, h: (h, 0, 0)),
    ],
    out_specs=pl.BlockSpec((R, F), lambda i, h: (i, h)),
    out_shape=jax.ShapeDtypeStruct((NROWS, 4 * F), jnp.float32),
)


def _e2_body(o_ref, b_ref, wc_ref, y_ref):
    g0 = o_ref[0, 0] + o_ref[1, 0]
    g1 = o_ref[0, 1] + o_ref[1, 1]
    feats = jnp.concatenate([g0[:, :FG], g1[:, :FG]], axis=1)
    den = jnp.maximum(g0[:, FG:FG + 1], 1e-30)
    h2 = feats / den + b_ref[0][None, :]
    y = jnp.sum(h2 * wc_ref[0][None, :], axis=1)
    y_ref[...] = jnp.broadcast_to(y[None, :], (8, R))


_e2_call = pl.pallas_call(
    _e2_body,
    grid=(NB,),
    in_specs=[
        pl.BlockSpec((NC, 2, R, DCOL), lambda i: (0, 0, i, 0)),
        pl.BlockSpec((1, F), lambda i: (0, 0)),
        pl.BlockSpec((1, F), lambda i: (0, 0)),
    ],
    out_specs=pl.BlockSpec((8, R), lambda i: (0, i)),
    out_shape=jax.ShapeDtypeStruct((8, NROWS), jnp.float32),
)

_a1_call = _make_a_call(4, 128)
_a2_call = _make_a_call(1, 512)
_sc4_call = _make_sc_call(4)
_sc1_call = _make_sc_call(1)


def _shift_vec(sh_raw):
    return jnp.broadcast_to(
        jnp.maximum(sh_raw[:, 0] + sh_raw[:, 8], 0.0)[:, None],
        (sh_raw.shape[0], 16))


@jax.jit
def kernel(x, edge_index, W1, att_src1, att_dst1, b1,
           W2, att_src2, att_dst2, b2, Wc, bc):
    n = x.shape[0]
    xp = jnp.pad(x, ((0, NROWS - n), (0, 0)))
    loop = jnp.arange(n, dtype=jnp.int32)
    src = jnp.concatenate([edge_index[0].astype(jnp.int32), loop])
    dst = jnp.concatenate([edge_index[1].astype(jnp.int32), loop])
    pad_e = E_PAD - src.shape[0]
    srcp = jnp.concatenate([src, jnp.full((pad_e,), n, jnp.int32)])
    dstp = jnp.concatenate([dst, jnp.zeros((pad_e,), jnp.int32)])

    tab1, as1, ad1, shraw1 = _a1_call(xp, W1, att_src1, att_dst1)
    o1 = _sc4_call(tab1.reshape(8 * NROWS, DCOL), srcp, dstp,
                   as1, ad1, _shift_vec(shraw1))
    h2in = _e1_call(o1, b1.reshape(4, 1, F))

    tab2, as2, ad2, shraw2 = _a2_call(h2in, W2, att_src2, att_dst2)
    o2 = _sc1_call(tab2.reshape(2 * NROWS, DCOL), srcp, dstp,
                   as2, ad2, _shift_vec(shraw2))
    y8 = _e2_call(o2, b2.reshape(1, F), Wc.reshape(1, F))
    return y8[0, :n] + bc[0]
